# Initial kernel scaffold; baseline (speedup 1.0000x reference)
import jax
import jax.numpy as jnp
from jax.experimental import pallas as pl


def kernel(z, Wp, bp, W0, b0, W1, b1, W2, b2, uw0, uw1, uw2, S0, S1, S2, ui0, ui1, ui2):
    raise NotImplementedError("write your pallas kernel here")


# speedup vs baseline: 2.3561x; 2.3561x over previous
import functools

import jax
import jax.numpy as jnp
from jax import lax
from jax.experimental import pallas as pl
from jax.experimental.pallas import tpu as pltpu
from jax.experimental.pallas import tpu_sc as plsc

SEQ = 16
B = 8
NC, NS = 2, 16
NW = NC * NS


def _mm(x, w, bm, bn):
    M, K = x.shape
    _, N = w.shape

    def body(xr, wr, outr):
        outr[...] = jnp.dot(xr[...], wr[...], preferred_element_type=jnp.float32)

    return pl.pallas_call(
        body,
        grid=(M // bm, N // bn),
        in_specs=[pl.BlockSpec((bm, K), lambda i, j: (i, 0)),
                  pl.BlockSpec((K, bn), lambda i, j: (0, j))],
        out_specs=pl.BlockSpec((bm, bn), lambda i, j: (i, j)),
        out_shape=jax.ShapeDtypeStruct((M, N), jnp.float32),
    )(x, w)


def _mm_nt(x, w, bias, bn):
    M, K = x.shape
    N = w.shape[0]

    def body(xr, wr, br, outr):
        acc = lax.dot_general(xr[...], wr[...], (((1,), (1,)), ((), ())),
                              preferred_element_type=jnp.float32)
        outr[...] = acc + br[...]

    return pl.pallas_call(
        body,
        grid=(N // bn,),
        in_specs=[pl.BlockSpec((M, K), lambda j: (0, 0)),
                  pl.BlockSpec((bn, K), lambda j: (j, 0)),
                  pl.BlockSpec((1, bn), lambda j: (0, j))],
        out_specs=pl.BlockSpec((M, bn), lambda j: (0, j)),
        out_shape=jax.ShapeDtypeStruct((M, N), jnp.float32),
    )(x, w, bias)


def _sc_pool(V, C, G):
    NG = V // G
    per_w = (B * NG) // NW
    mesh = plsc.VectorSubcoreMesh(core_axis_name="c", subcore_axis_name="s",
                                  num_cores=NC, num_subcores=NS)

    @functools.partial(
        pl.kernel, mesh=mesh,
        out_type=jax.ShapeDtypeStruct((B, V, C), jnp.float32),
        scratch_types=[
            pltpu.VMEM((3, G), jnp.int32),
            pltpu.VMEM((3, G), jnp.float32),
            pltpu.VMEM((3, G, C), jnp.float32),
            pltpu.SemaphoreType.DMA,
        ],
    )
    def k(src, idx, w, out, idxv, wv, rows, sem):
        wid = lax.axis_index("s") * NC + lax.axis_index("c")

        def grp(i, carry):
            b = i // NG
            g = i % NG
            pltpu.sync_copy(idx.at[b, g], idxv)
            pltpu.sync_copy(w.at[g], wv)
            cps = [pltpu.async_copy(src.at[idxv.at[t]], rows.at[t], sem)
                   for t in range(3)]
            for cp in cps:
                cp.wait()

            def rowchunk(rc, carry2):
                wr = [wv[t, pl.ds(rc * 16, 16)] for t in range(3)]
                for j in range(16):
                    r = rc * 16 + j
                    w0, w1, w2 = wr[0][j], wr[1][j], wr[2][j]

                    def cchunk(cc, carry3):
                        cs = pl.ds(cc * 16, 16)
                        acc = (w0 * rows[0, r, cs] + w1 * rows[1, r, cs]
                               + w2 * rows[2, r, cs])
                        rows[0, r, cs] = acc
                        return carry3

                    lax.fori_loop(0, C // 16, cchunk, 0)
                return carry2

            lax.fori_loop(0, G // 16, rowchunk, 0)
            pltpu.sync_copy(rows.at[0], out.at[b, pl.ds(g * G, G)])
            return carry

        lax.fori_loop(wid * per_w, (wid + 1) * per_w, grp, 0)

    return k


def _sc_comp(V, G, T):
    C = 16
    NG = V // G
    per_w = (B * NG) // NW
    mesh = plsc.VectorSubcoreMesh(core_axis_name="c", subcore_axis_name="s",
                                  num_cores=NC, num_subcores=NS)

    @functools.partial(
        pl.kernel, mesh=mesh,
        out_type=jax.ShapeDtypeStruct((B, V, C), jnp.float32),
        compiler_params=pltpu.CompilerParams(use_tc_tiling_on_sc=False),
        scratch_types=[
            pltpu.VMEM((T, G), jnp.int32),
            pltpu.VMEM((T, G), jnp.float32),
            pltpu.VMEM((T, G, C), jnp.float32),
            pltpu.VMEM((C,), jnp.float32),
            pltpu.SemaphoreType.DMA,
        ],
    )
    def k(src, idx, w, bias, out, idxv, wv, rows, biasv, sem):
        wid = lax.axis_index("s") * NC + lax.axis_index("c")
        pltpu.sync_copy(bias, biasv)

        def grp(i, carry):
            b = i // NG
            g = i % NG
            pltpu.sync_copy(idx.at[b, g], idxv)
            pltpu.sync_copy(w.at[g], wv)
            cps = [pltpu.async_copy(src.at[idxv.at[t]], rows.at[t], sem)
                   for t in range(T)]
            for cp in cps:
                cp.wait()

            def rowchunk(rc, carry2):
                accs = [biasv[:] for _ in range(16)]
                for t in range(T):
                    wreg = wv[t, pl.ds(rc * 16, 16)]
                    for j in range(16):
                        accs[j] = accs[j] + wreg[j] * rows[t, rc * 16 + j, :]
                for j in range(16):
                    rows[0, rc * 16 + j, :] = accs[j]
                return carry2

            lax.fori_loop(0, G // 16, rowchunk, 0)
            pltpu.sync_copy(rows.at[0], out.at[b, pl.ds(g * G, G)])
            return carry

        lax.fori_loop(wid * per_w, (wid + 1) * per_w, grp, 0)

    return k


def _sc_spiral(V, C, G, act):
    NG = V // G
    per_w = (B * NG) // NW
    mesh = plsc.VectorSubcoreMesh(core_axis_name="c", subcore_axis_name="s",
                                  num_cores=NC, num_subcores=NS)

    @functools.partial(
        pl.kernel, mesh=mesh,
        out_type=jax.ShapeDtypeStruct((B, V, C), jnp.float32),
        scratch_types=[
            pltpu.VMEM((SEQ, G), jnp.int32),
            pltpu.VMEM((SEQ, G, C), jnp.float32),
            pltpu.VMEM((C,), jnp.float32),
            pltpu.SemaphoreType.DMA,
        ],
    )
    def k(src, idx, bias, out, idxv, rows, biasv, sem):
        wid = lax.axis_index("s") * NC + lax.axis_index("c")
        pltpu.sync_copy(bias, biasv)

        def grp(i, carry):
            b = i // NG
            g = i % NG
            pltpu.sync_copy(idx.at[b, g], idxv)
            cps = [pltpu.async_copy(src.at[idxv.at[t]], rows.at[t], sem)
                   for t in range(SEQ)]
            for cp in cps:
                cp.wait()

            def row(r, carry2):
                def cchunk(cc, carry3):
                    cs = pl.ds(cc * 16, 16)
                    acc = rows[0, r, cs]
                    for t in range(1, SEQ):
                        acc = acc + rows[t, r, cs]
                    acc = acc + biasv[cs]
                    if act:
                        acc = jnp.where(acc > 0, acc, jnp.exp(acc) - 1.0)
                    rows[0, r, cs] = acc
                    return carry3

                lax.fori_loop(0, C // 16, cchunk, 0)
                return carry2

            lax.fori_loop(0, G, row, 0)
            pltpu.sync_copy(rows.at[0], out.at[b, pl.ds(g * G, G)])
            return carry

        lax.fori_loop(wid * per_w, (wid + 1) * per_w, grp, 0)

    return k


def _prep_pool(ui, uw, R, G):
    V = ui.shape[0]
    NG = V // G
    idx = ui.astype(jnp.int32)[None] + (jnp.arange(B, dtype=jnp.int32) * R)[:, None, None]
    idxg = idx.reshape(B, NG, G, 3).transpose(0, 1, 3, 2)
    wg = uw.reshape(NG, G, 3).transpose(0, 2, 1)
    return idxg, wg


def _prep_spiral(S, G):
    V = S.shape[0]
    NG = V // G
    base = (S * SEQ + jnp.arange(SEQ, dtype=jnp.int32)[None, :]).astype(jnp.int32)
    idx = base[None] + (jnp.arange(B, dtype=jnp.int32) * (V * SEQ))[:, None, None]
    return idx.reshape(B, NG, G, SEQ).transpose(0, 1, 3, 2)


def _prep_comp(S, ui, uw, R, G):
    V = S.shape[0]
    NG = V // G
    su = S.astype(jnp.int32)
    base = (ui.astype(jnp.int32)[su] * SEQ
            + jnp.arange(SEQ, dtype=jnp.int32)[None, :, None])
    cw = uw[su]
    T = 3 * SEQ
    idx = base.reshape(V, T)[None] + (jnp.arange(B, dtype=jnp.int32) * (R * SEQ))[:, None, None]
    idxg = idx.reshape(B, NG, G, T).transpose(0, 1, 3, 2)
    wg = cw.reshape(NG, G, T).transpose(0, 2, 1)
    return idxg, wg


def _w_all(W, C_in, C_out, pad_to=None):
    P = pad_to or C_out
    wa = W.reshape(C_out, SEQ, C_in).transpose(2, 1, 0)
    if P != C_out:
        wa = jnp.pad(wa, ((0, 0), (0, 0), (0, P - C_out)))
    return wa.reshape(C_in, SEQ * P)


def kernel(z, Wp, bp, W0, b0, W1, b1, W2, b2, uw0, uw1, uw2,
           S0, S1, S2, ui0, ui1, ui2):
    x0 = _mm_nt(z, Wp, bp.reshape(1, -1), 4096).reshape(B * 256, 256)

    idxA, wA = _prep_pool(ui2, uw2, 256, 64)
    pA = _sc_pool(1024, 256, 64)(x0, idxA, wA).reshape(B * 1024, 256)
    hA = _mm(pA, _w_all(W0, 256, 256), 1024, 1024).reshape(B * 1024 * SEQ, 256)
    yA = _sc_spiral(1024, 256, 16, True)(hA, _prep_spiral(S2, 16), b0)

    idxB, wB = _prep_pool(ui1, uw1, 1024, 64)
    pB = _sc_pool(4096, 256, 64)(yA.reshape(B * 1024, 256), idxB, wB).reshape(B * 4096, 256)
    hB = _mm(pB, _w_all(W1, 256, 128), 1024, 1024).reshape(B * 4096 * SEQ, 128)
    yB = _sc_spiral(4096, 128, 32, True)(hB, _prep_spiral(S1, 32), b1)

    hC = _mm(yB.reshape(B * 4096, 128), _w_all(W2, 128, 3, pad_to=16),
             2048, 256).reshape(B * 4096 * SEQ, 16)
    idxC, wC = _prep_comp(S0, ui0, uw0, 4096, 128)
    b2p = jnp.pad(b2, (0, 13))
    yC = _sc_comp(16384, 128, 48)(hC, idxC, wC, b2p)

    return yC[..., :3]


# speedup vs baseline: 2.4305x; 1.0316x over previous
import functools

import jax
import jax.numpy as jnp
from jax import lax
from jax.experimental import pallas as pl
from jax.experimental.pallas import tpu as pltpu
from jax.experimental.pallas import tpu_sc as plsc

SEQ = 16
B = 8
NC, NS = 2, 16
NW = NC * NS


def _mm(x, w, bm, bn):
    M, K = x.shape
    _, N = w.shape

    def body(xr, wr, outr):
        outr[...] = jnp.dot(xr[...], wr[...], preferred_element_type=jnp.float32)

    return pl.pallas_call(
        body,
        grid=(M // bm, N // bn),
        in_specs=[pl.BlockSpec((bm, K), lambda i, j: (i, 0)),
                  pl.BlockSpec((K, bn), lambda i, j: (0, j))],
        out_specs=pl.BlockSpec((bm, bn), lambda i, j: (i, j)),
        out_shape=jax.ShapeDtypeStruct((M, N), jnp.float32),
    )(x, w)


def _mm_smajor(x, w, bm):
    M, K = x.shape
    C = w.shape[1] // SEQ

    def body(xr, wr, outr):
        outr[0] = jnp.dot(xr[...], wr[...], preferred_element_type=jnp.float32)

    return pl.pallas_call(
        body,
        grid=(M // bm, SEQ),
        in_specs=[pl.BlockSpec((bm, K), lambda i, s: (i, 0)),
                  pl.BlockSpec((K, C), lambda i, s: (0, s))],
        out_specs=pl.BlockSpec((1, bm, C), lambda i, s: (s, i, 0)),
        out_shape=jax.ShapeDtypeStruct((SEQ, M, C), jnp.float32),
    )(x, w)


def _mm_nt(x, w, bias, bn):
    M, K = x.shape
    N = w.shape[0]

    def body(xr, wr, br, outr):
        acc = lax.dot_general(xr[...], wr[...], (((1,), (1,)), ((), ())),
                              preferred_element_type=jnp.float32)
        outr[...] = acc + br[...]

    return pl.pallas_call(
        body,
        grid=(N // bn,),
        in_specs=[pl.BlockSpec((M, K), lambda j: (0, 0)),
                  pl.BlockSpec((bn, K), lambda j: (j, 0)),
                  pl.BlockSpec((1, bn), lambda j: (0, j))],
        out_specs=pl.BlockSpec((M, bn), lambda j: (0, j)),
        out_shape=jax.ShapeDtypeStruct((M, N), jnp.float32),
    )(x, w, bias)


def _sc_pool(V, C, G):
    NG = V // G
    per_w = (B * NG) // NW
    mesh = plsc.VectorSubcoreMesh(core_axis_name="c", subcore_axis_name="s",
                                  num_cores=NC, num_subcores=NS)

    @functools.partial(
        pl.kernel, mesh=mesh,
        out_type=jax.ShapeDtypeStruct((B, V, C), jnp.float32),
        scratch_types=[
            pltpu.VMEM((3, G), jnp.int32),
            pltpu.VMEM((3, G), jnp.float32),
            pltpu.VMEM((3, G, C), jnp.float32),
            pltpu.SemaphoreType.DMA,
        ],
    )
    def k(src, idx, w, out, idxv, wv, rows, sem):
        wid = lax.axis_index("s") * NC + lax.axis_index("c")

        def grp(i, carry):
            b = i // NG
            g = i % NG
            pltpu.sync_copy(idx.at[b, g], idxv)
            pltpu.sync_copy(w.at[g], wv)
            cps = [pltpu.async_copy(src.at[idxv.at[t]], rows.at[t], sem)
                   for t in range(3)]
            for cp in cps:
                cp.wait()

            def rowchunk(rc, carry2):
                wr = [wv[t, pl.ds(rc * 16, 16)] for t in range(3)]
                for j in range(16):
                    r = rc * 16 + j
                    w0, w1, w2 = wr[0][j], wr[1][j], wr[2][j]

                    def cchunk(cc, carry3):
                        cs = pl.ds(cc * 16, 16)
                        acc = (w0 * rows[0, r, cs] + w1 * rows[1, r, cs]
                               + w2 * rows[2, r, cs])
                        rows[0, r, cs] = acc
                        return carry3

                    lax.fori_loop(0, C // 16, cchunk, 0)
                return carry2

            lax.fori_loop(0, G // 16, rowchunk, 0)
            pltpu.sync_copy(rows.at[0], out.at[b, pl.ds(g * G, G)])
            return carry

        lax.fori_loop(wid * per_w, (wid + 1) * per_w, grp, 0)

    return k


def _sc_comp(V, G, T):
    C = 16
    NG = V // G
    per_w = (B * NG) // NW
    mesh = plsc.VectorSubcoreMesh(core_axis_name="c", subcore_axis_name="s",
                                  num_cores=NC, num_subcores=NS)

    @functools.partial(
        pl.kernel, mesh=mesh,
        out_type=jax.ShapeDtypeStruct((B, V, C), jnp.float32),
        compiler_params=pltpu.CompilerParams(use_tc_tiling_on_sc=False),
        scratch_types=[
            pltpu.VMEM((T, G), jnp.int32),
            pltpu.VMEM((T, G), jnp.float32),
            pltpu.VMEM((T, G, C), jnp.float32),
            pltpu.VMEM((C,), jnp.float32),
            pltpu.SemaphoreType.DMA,
        ],
    )
    def k(src, idx, w, bias, out, idxv, wv, rows, biasv, sem):
        wid = lax.axis_index("s") * NC + lax.axis_index("c")
        pltpu.sync_copy(bias, biasv)

        def grp(i, carry):
            b = i // NG
            g = i % NG
            pltpu.sync_copy(idx.at[b, g], idxv)
            pltpu.sync_copy(w.at[g], wv)
            cps = [pltpu.async_copy(src.at[idxv.at[t]], rows.at[t], sem)
                   for t in range(T)]
            for cp in cps:
                cp.wait()

            def rowchunk(rc, carry2):
                accs = [biasv[:] for _ in range(16)]
                for t in range(T):
                    wreg = wv[t, pl.ds(rc * 16, 16)]
                    for j in range(16):
                        accs[j] = accs[j] + wreg[j] * rows[t, rc * 16 + j, :]
                for j in range(16):
                    rows[0, rc * 16 + j, :] = accs[j]
                return carry2

            lax.fori_loop(0, G // 16, rowchunk, 0)
            pltpu.sync_copy(rows.at[0], out.at[b, pl.ds(g * G, G)])
            return carry

        lax.fori_loop(wid * per_w, (wid + 1) * per_w, grp, 0)

    return k


def _sc_spiral(V, C, G, act):
    NG = V // G
    per_w = (B * NG) // NW
    mesh = plsc.VectorSubcoreMesh(core_axis_name="c", subcore_axis_name="s",
                                  num_cores=NC, num_subcores=NS)

    @functools.partial(
        pl.kernel, mesh=mesh,
        out_type=jax.ShapeDtypeStruct((B, V, C), jnp.float32),
        scratch_types=[
            pltpu.VMEM((SEQ, G), jnp.int32),
            pltpu.VMEM((SEQ, G, C), jnp.float32),
            pltpu.VMEM((C,), jnp.float32),
            pltpu.SemaphoreType.DMA,
        ],
    )
    def k(src, idx, bias, out, idxv, rows, biasv, sem):
        wid = lax.axis_index("s") * NC + lax.axis_index("c")
        pltpu.sync_copy(bias, biasv)

        def grp(i, carry):
            b = i // NG
            g = i % NG
            pltpu.sync_copy(idx.at[b, g], idxv)
            cps = [pltpu.async_copy(src.at[idxv.at[t]], rows.at[t], sem)
                   for t in range(SEQ)]
            for cp in cps:
                cp.wait()

            def row(r, carry2):
                def cchunk(cc, carry3):
                    cs = pl.ds(cc * 16, 16)
                    acc = rows[0, r, cs]
                    for t in range(1, SEQ):
                        acc = acc + rows[t, r, cs]
                    acc = acc + biasv[cs]
                    if act:
                        acc = jnp.where(acc > 0, acc, jnp.exp(acc) - 1.0)
                    rows[0, r, cs] = acc
                    return carry3

                lax.fori_loop(0, C // 16, cchunk, 0)
                return carry2

            lax.fori_loop(0, G, row, 0)
            pltpu.sync_copy(rows.at[0], out.at[b, pl.ds(g * G, G)])
            return carry

        lax.fori_loop(wid * per_w, (wid + 1) * per_w, grp, 0)

    return k


def _prep_pool(ui, uw, R, G):
    V = ui.shape[0]
    NG = V // G
    idx = ui.astype(jnp.int32)[None] + (jnp.arange(B, dtype=jnp.int32) * R)[:, None, None]
    idxg = idx.reshape(B, NG, G, 3).transpose(0, 1, 3, 2)
    wg = uw.reshape(NG, G, 3).transpose(0, 2, 1)
    return idxg, wg


def _prep_spiral(S, G):
    V = S.shape[0]
    NG = V // G
    base = (S + (jnp.arange(SEQ, dtype=jnp.int32) * (B * V))[None, :]).astype(jnp.int32)
    idx = base[None] + (jnp.arange(B, dtype=jnp.int32) * V)[:, None, None]
    return idx.reshape(B, NG, G, SEQ).transpose(0, 1, 3, 2)


def _prep_comp(S, ui, uw, R, G):
    V = S.shape[0]
    NG = V // G
    su = S.astype(jnp.int32)
    base = (ui.astype(jnp.int32)[su] * SEQ
            + jnp.arange(SEQ, dtype=jnp.int32)[None, :, None])
    cw = uw[su]
    T = 3 * SEQ
    idx = base.reshape(V, T)[None] + (jnp.arange(B, dtype=jnp.int32) * (R * SEQ))[:, None, None]
    idxg = idx.reshape(B, NG, G, T).transpose(0, 1, 3, 2)
    wg = cw.reshape(NG, G, T).transpose(0, 2, 1)
    return idxg, wg


def _w_all(W, C_in, C_out, pad_to=None):
    P = pad_to or C_out
    wa = W.reshape(C_out, SEQ, C_in).transpose(2, 1, 0)
    if P != C_out:
        wa = jnp.pad(wa, ((0, 0), (0, 0), (0, P - C_out)))
    return wa.reshape(C_in, SEQ * P)


def kernel(z, Wp, bp, W0, b0, W1, b1, W2, b2, uw0, uw1, uw2,
           S0, S1, S2, ui0, ui1, ui2):
    x0 = _mm_nt(z, Wp, bp.reshape(1, -1), 4096).reshape(B * 256, 256)

    idxA, wA = _prep_pool(ui2, uw2, 256, 64)
    pA = _sc_pool(1024, 256, 64)(x0, idxA, wA).reshape(B * 1024, 256)
    hA = _mm_smajor(pA, _w_all(W0, 256, 256), 1024).reshape(SEQ * B * 1024, 256)
    yA = _sc_spiral(1024, 256, 16, True)(hA, _prep_spiral(S2, 16), b0)

    idxB, wB = _prep_pool(ui1, uw1, 1024, 64)
    pB = _sc_pool(4096, 256, 64)(yA.reshape(B * 1024, 256), idxB, wB).reshape(B * 4096, 256)
    hB = _mm_smajor(pB, _w_all(W1, 256, 128), 1024).reshape(SEQ * B * 4096, 128)
    yB = _sc_spiral(4096, 128, 32, True)(hB, _prep_spiral(S1, 32), b1)

    hC = _mm(yB.reshape(B * 4096, 128), _w_all(W2, 128, 3, pad_to=16),
             2048, 256).reshape(B * 4096 * SEQ, 16)
    idxC, wC = _prep_comp(S0, ui0, uw0, 4096, 128)
    b2p = jnp.pad(b2, (0, 13))
    yC = _sc_comp(16384, 128, 48)(hC, idxC, wC, b2p)

    return yC[..., :3]


# speedup vs baseline: 4.3865x; 1.8048x over previous
import functools

import jax
import jax.numpy as jnp
from jax import lax
from jax.experimental import pallas as pl
from jax.experimental.pallas import tpu as pltpu
from jax.experimental.pallas import tpu_sc as plsc

SEQ = 16
B = 8
NC, NS = 2, 16
NW = NC * NS


def _mm(x, w, bm, bn):
    M, K = x.shape
    _, N = w.shape

    def body(xr, wr, outr):
        outr[...] = jnp.dot(xr[...], wr[...], preferred_element_type=jnp.float32)

    return pl.pallas_call(
        body,
        grid=(M // bm, N // bn),
        in_specs=[pl.BlockSpec((bm, K), lambda i, j: (i, 0)),
                  pl.BlockSpec((K, bn), lambda i, j: (0, j))],
        out_specs=pl.BlockSpec((bm, bn), lambda i, j: (i, j)),
        out_shape=jax.ShapeDtypeStruct((M, N), jnp.float32),
    )(x, w)


def _mm_smajor(x, w, bm):
    M, K = x.shape
    C = w.shape[1] // SEQ

    def body(xr, wr, outr):
        outr[0] = jnp.dot(xr[...], wr[...], preferred_element_type=jnp.float32)

    return pl.pallas_call(
        body,
        grid=(M // bm, SEQ),
        in_specs=[pl.BlockSpec((bm, K), lambda i, s: (i, 0)),
                  pl.BlockSpec((K, C), lambda i, s: (0, s))],
        out_specs=pl.BlockSpec((1, bm, C), lambda i, s: (s, i, 0)),
        out_shape=jax.ShapeDtypeStruct((SEQ, M, C), jnp.float32),
    )(x, w)


def _mm_nt(x, w, bias, bn):
    M, K = x.shape
    N = w.shape[0]

    def body(xr, wr, br, outr):
        acc = lax.dot_general(xr[...], wr[...], (((1,), (1,)), ((), ())),
                              preferred_element_type=jnp.float32)
        outr[...] = acc + br[...]

    return pl.pallas_call(
        body,
        grid=(N // bn,),
        in_specs=[pl.BlockSpec((M, K), lambda j: (0, 0)),
                  pl.BlockSpec((bn, K), lambda j: (j, 0)),
                  pl.BlockSpec((1, bn), lambda j: (0, j))],
        out_specs=pl.BlockSpec((M, bn), lambda j: (0, j)),
        out_shape=jax.ShapeDtypeStruct((M, N), jnp.float32),
    )(x, w, bias)


def _sc_pool(V, C, G):
    NG = V // G
    per_w = (B * NG) // NW
    mesh = plsc.VectorSubcoreMesh(core_axis_name="c", subcore_axis_name="s",
                                  num_cores=NC, num_subcores=NS)

    @functools.partial(
        pl.kernel, mesh=mesh,
        out_type=jax.ShapeDtypeStruct((B, V, C), jnp.float32),
        scratch_types=[
            pltpu.VMEM((3, G), jnp.int32),
            pltpu.VMEM((3, G), jnp.float32),
            pltpu.VMEM((3, G, C), jnp.float32),
            pltpu.SemaphoreType.DMA,
        ],
    )
    def k(src, idx, w, out, idxv, wv, rows, sem):
        wid = lax.axis_index("s") * NC + lax.axis_index("c")

        def grp(i, carry):
            b = i // NG
            g = i % NG
            pltpu.sync_copy(idx.at[b, g], idxv)
            pltpu.sync_copy(w.at[g], wv)
            cps = [pltpu.async_copy(src.at[idxv.at[t]], rows.at[t], sem)
                   for t in range(3)]
            for cp in cps:
                cp.wait()

            def rowchunk(rc, carry2):
                wr = [wv[t, pl.ds(rc * 16, 16)] for t in range(3)]
                for j in range(16):
                    r = rc * 16 + j
                    w0, w1, w2 = wr[0][j], wr[1][j], wr[2][j]

                    def cchunk(cc, carry3):
                        cs = pl.ds(cc * 16, 16)
                        acc = (w0 * rows[0, r, cs] + w1 * rows[1, r, cs]
                               + w2 * rows[2, r, cs])
                        rows[0, r, cs] = acc
                        return carry3

                    lax.fori_loop(0, C // 16, cchunk, 0)
                return carry2

            lax.fori_loop(0, G // 16, rowchunk, 0)
            pltpu.sync_copy(rows.at[0], out.at[b, pl.ds(g * G, G)])
            return carry

        lax.fori_loop(wid * per_w, (wid + 1) * per_w, grp, 0)

    return k


def _sc_comp(V, RR, G):
    C = 16
    T = 3 * SEQ
    NG = V // G
    gpw = NG // NW
    mesh = plsc.VectorSubcoreMesh(core_axis_name="c", subcore_axis_name="s",
                                  num_cores=NC, num_subcores=NS)

    @functools.partial(
        pl.kernel, mesh=mesh,
        out_type=jax.ShapeDtypeStruct((B, V, C), jnp.float32),
        compiler_params=pltpu.CompilerParams(use_tc_tiling_on_sc=False,
                                             needs_layout_passes=False),
        scratch_types=[
            pltpu.VMEM((SEQ, G), jnp.int32),
            pltpu.VMEM((SEQ, G, 16), jnp.float32),
            pltpu.VMEM((T, G), jnp.int32),
            pltpu.VMEM((T, G), jnp.int32),
            pltpu.VMEM((T, G), jnp.float32),
            pltpu.VMEM((T, G, C), jnp.float32),
            pltpu.VMEM((C,), jnp.float32),
            pltpu.SemaphoreType.DMA,
        ],
    )
    def k(src, sg, uiw, bias, out, sidxv, uiwg, idxb, idxv, wv, rows, biasv, sem):
        wid = lax.axis_index("s") * NC + lax.axis_index("c")
        pltpu.sync_copy(bias, biasv)

        def grp(gi, carry):
            pltpu.sync_copy(sg.at[gi], sidxv)
            cps = [pltpu.async_copy(uiw.at[sidxv.at[s]], uiwg.at[s], sem)
                   for s in range(SEQ)]
            for cp in cps:
                cp.wait()

            def compose(rc, carry2):
                rids = lax.iota(jnp.int32, 16) + rc * 16
                for s in range(SEQ):
                    svec = jnp.full((16,), s, jnp.int32)
                    for kk in range(3):
                        t = s * 3 + kk
                        iv = plsc.load_gather(
                            uiwg, [svec, rids, jnp.full((16,), kk, jnp.int32)])
                        idxb[t, pl.ds(rc * 16, 16)] = iv.astype(jnp.int32) + s
                        wvv = plsc.load_gather(
                            uiwg, [svec, rids, jnp.full((16,), 4 + kk, jnp.int32)])
                        wv[t, pl.ds(rc * 16, 16)] = wvv
                return carry2

            lax.fori_loop(0, G // 16, compose, 0)

            def batch(b, carry2):
                def off(rc, carry3):
                    cs = pl.ds(rc * 16, 16)
                    for t in range(T):
                        idxv[t, cs] = idxb[t, cs] + b * (RR * SEQ)
                    return carry3

                lax.fori_loop(0, G // 16, off, 0)
                cps2 = [pltpu.async_copy(src.at[idxv.at[t]], rows.at[t], sem)
                        for t in range(T)]
                for cp in cps2:
                    cp.wait()

                def rowchunk(rc, carry3):
                    accs = [biasv[:] for _ in range(16)]
                    for t in range(T):
                        wreg = wv[t, pl.ds(rc * 16, 16)]
                        for j in range(16):
                            accs[j] = accs[j] + wreg[j] * rows[t, rc * 16 + j, :]
                    for j in range(16):
                        rows[0, rc * 16 + j, :] = accs[j]
                    return carry3

                lax.fori_loop(0, G // 16, rowchunk, 0)
                pltpu.sync_copy(rows.at[0], out.at[b, pl.ds(gi * G, G)])
                return carry2

            lax.fori_loop(0, B, batch, 0)
            return carry

        lax.fori_loop(wid * gpw, (wid + 1) * gpw, grp, 0)

    return k


def _sc_spiral(V, C, G, act):
    NG = V // G
    per_w = (B * NG) // NW
    mesh = plsc.VectorSubcoreMesh(core_axis_name="c", subcore_axis_name="s",
                                  num_cores=NC, num_subcores=NS)

    @functools.partial(
        pl.kernel, mesh=mesh,
        out_type=jax.ShapeDtypeStruct((B, V, C), jnp.float32),
        scratch_types=[
            pltpu.VMEM((SEQ, G), jnp.int32),
            pltpu.VMEM((SEQ, G, C), jnp.float32),
            pltpu.VMEM((C,), jnp.float32),
            pltpu.SemaphoreType.DMA,
        ],
    )
    def k(src, idx, bias, out, idxv, rows, biasv, sem):
        wid = lax.axis_index("s") * NC + lax.axis_index("c")
        pltpu.sync_copy(bias, biasv)

        def grp(i, carry):
            b = i // NG
            g = i % NG
            pltpu.sync_copy(idx.at[b, g], idxv)
            cps = [pltpu.async_copy(src.at[idxv.at[t]], rows.at[t], sem)
                   for t in range(SEQ)]
            for cp in cps:
                cp.wait()

            def row(r, carry2):
                def cchunk(cc, carry3):
                    cs = pl.ds(cc * 16, 16)
                    acc = rows[0, r, cs]
                    for t in range(1, SEQ):
                        acc = acc + rows[t, r, cs]
                    acc = acc + biasv[cs]
                    if act:
                        acc = jnp.where(acc > 0, acc, jnp.exp(acc) - 1.0)
                    rows[0, r, cs] = acc
                    return carry3

                lax.fori_loop(0, C // 16, cchunk, 0)
                return carry2

            lax.fori_loop(0, G, row, 0)
            pltpu.sync_copy(rows.at[0], out.at[b, pl.ds(g * G, G)])
            return carry

        lax.fori_loop(wid * per_w, (wid + 1) * per_w, grp, 0)

    return k


def _prep_pool(ui, uw, R, G):
    V = ui.shape[0]
    NG = V // G
    idx = ui.astype(jnp.int32)[None] + (jnp.arange(B, dtype=jnp.int32) * R)[:, None, None]
    idxg = idx.reshape(B, NG, G, 3).transpose(0, 1, 3, 2)
    wg = uw.reshape(NG, G, 3).transpose(0, 2, 1)
    return idxg, wg


def _prep_spiral(S, G):
    V = S.shape[0]
    NG = V // G
    base = (S + (jnp.arange(SEQ, dtype=jnp.int32) * (B * V))[None, :]).astype(jnp.int32)
    idx = base[None] + (jnp.arange(B, dtype=jnp.int32) * V)[:, None, None]
    return idx.reshape(B, NG, G, SEQ).transpose(0, 1, 3, 2)


def _prep_comp(S, ui, uw, G):
    V, NG = S.shape[0], S.shape[0] // G
    ui16 = (ui.astype(jnp.int32) * SEQ).astype(jnp.float32)
    z1 = jnp.zeros((ui.shape[0], 1), jnp.float32)
    uiw = jnp.concatenate(
        [ui16, z1, uw, jnp.zeros((ui.shape[0], 9), jnp.float32)], axis=1)
    sg = S.astype(jnp.int32).reshape(NG, G, SEQ).transpose(0, 2, 1)
    return sg, uiw


def _w_all(W, C_in, C_out, pad_to=None):
    P = pad_to or C_out
    wa = W.reshape(C_out, SEQ, C_in).transpose(2, 1, 0)
    if P != C_out:
        wa = jnp.pad(wa, ((0, 0), (0, 0), (0, P - C_out)))
    return wa.reshape(C_in, SEQ * P)


def kernel(z, Wp, bp, W0, b0, W1, b1, W2, b2, uw0, uw1, uw2,
           S0, S1, S2, ui0, ui1, ui2):
    x0 = _mm_nt(z, Wp, bp.reshape(1, -1), 4096).reshape(B * 256, 256)

    idxA, wA = _prep_pool(ui2, uw2, 256, 64)
    pA = _sc_pool(1024, 256, 64)(x0, idxA, wA).reshape(B * 1024, 256)
    hA = _mm_smajor(pA, _w_all(W0, 256, 256), 1024).reshape(SEQ * B * 1024, 256)
    yA = _sc_spiral(1024, 256, 16, True)(hA, _prep_spiral(S2, 16), b0)

    idxB, wB = _prep_pool(ui1, uw1, 1024, 64)
    pB = _sc_pool(4096, 256, 64)(yA.reshape(B * 1024, 256), idxB, wB).reshape(B * 4096, 256)
    hB = _mm_smajor(pB, _w_all(W1, 256, 128), 1024).reshape(SEQ * B * 4096, 128)
    yB = _sc_spiral(4096, 128, 32, True)(hB, _prep_spiral(S1, 32), b1)

    hC = _mm(yB.reshape(B * 4096, 128), _w_all(W2, 128, 3, pad_to=16),
             2048, 256).reshape(B * 4096 * SEQ, 16)
    sgC, uiwC = _prep_comp(S0, ui0, uw0, 64)
    b2p = jnp.pad(b2, (0, 13))
    yC = _sc_comp(16384, 4096, 64)(hC, sgC, uiwC, b2p)

    return yC[..., :3]


# speedup vs baseline: 5.4019x; 1.2315x over previous
import functools

import jax
import jax.numpy as jnp
from jax import lax
from jax.experimental import pallas as pl
from jax.experimental.pallas import tpu as pltpu
from jax.experimental.pallas import tpu_sc as plsc

SEQ = 16
B = 8
NC, NS = 2, 16
NW = NC * NS


def _mm(x, w, bm, bn):
    M, K = x.shape
    _, N = w.shape

    def body(xr, wr, outr):
        outr[...] = jnp.dot(xr[...], wr[...], preferred_element_type=jnp.float32)

    return pl.pallas_call(
        body,
        grid=(M // bm, N // bn),
        in_specs=[pl.BlockSpec((bm, K), lambda i, j: (i, 0)),
                  pl.BlockSpec((K, bn), lambda i, j: (0, j))],
        out_specs=pl.BlockSpec((bm, bn), lambda i, j: (i, j)),
        out_shape=jax.ShapeDtypeStruct((M, N), jnp.float32),
    )(x, w)


def _mm_smajor(x, w, bm):
    M, K = x.shape
    C = w.shape[1] // SEQ

    def body(xr, wr, outr):
        outr[0] = jnp.dot(xr[...], wr[...], preferred_element_type=jnp.float32)

    return pl.pallas_call(
        body,
        grid=(M // bm, SEQ),
        in_specs=[pl.BlockSpec((bm, K), lambda i, s: (i, 0)),
                  pl.BlockSpec((K, C), lambda i, s: (0, s))],
        out_specs=pl.BlockSpec((1, bm, C), lambda i, s: (s, i, 0)),
        out_shape=jax.ShapeDtypeStruct((SEQ, M, C), jnp.float32),
    )(x, w)


def _mm_nt(x, w, bias, bn):
    M, K = x.shape
    N = w.shape[0]

    def body(xr, wr, br, outr):
        acc = lax.dot_general(xr[...], wr[...], (((1,), (1,)), ((), ())),
                              preferred_element_type=jnp.float32)
        outr[...] = acc + br[...]

    return pl.pallas_call(
        body,
        grid=(N // bn,),
        in_specs=[pl.BlockSpec((M, K), lambda j: (0, 0)),
                  pl.BlockSpec((bn, K), lambda j: (j, 0)),
                  pl.BlockSpec((1, bn), lambda j: (0, j))],
        out_specs=pl.BlockSpec((M, bn), lambda j: (0, j)),
        out_shape=jax.ShapeDtypeStruct((M, N), jnp.float32),
    )(x, w, bias)


def _sc_pool(V, C, G):
    NG = V // G
    per_w = (B * NG) // NW
    mesh = plsc.VectorSubcoreMesh(core_axis_name="c", subcore_axis_name="s",
                                  num_cores=NC, num_subcores=NS)

    @functools.partial(
        pl.kernel, mesh=mesh,
        out_type=jax.ShapeDtypeStruct((B, V, C), jnp.float32),
        scratch_types=[
            pltpu.VMEM((3, G), jnp.int32),
            pltpu.VMEM((3, G), jnp.float32),
            pltpu.VMEM((3, G, C), jnp.float32),
            pltpu.SemaphoreType.DMA,
        ],
    )
    def k(src, idx, w, out, idxv, wv, rows, sem):
        wid = lax.axis_index("s") * NC + lax.axis_index("c")

        def grp(i, carry):
            b = i // NG
            g = i % NG
            pltpu.sync_copy(idx.at[b, g], idxv)
            pltpu.sync_copy(w.at[g], wv)
            cps = [pltpu.async_copy(src.at[idxv.at[t]], rows.at[t], sem)
                   for t in range(3)]
            for cp in cps:
                cp.wait()

            def rowchunk(rc, carry2):
                wr = [wv[t, pl.ds(rc * 16, 16)] for t in range(3)]
                for j in range(16):
                    r = rc * 16 + j
                    w0, w1, w2 = wr[0][j], wr[1][j], wr[2][j]

                    def cchunk(cc, carry3):
                        cs = pl.ds(cc * 16, 16)
                        acc = (w0 * rows[0, r, cs] + w1 * rows[1, r, cs]
                               + w2 * rows[2, r, cs])
                        rows[0, r, cs] = acc
                        return carry3

                    lax.fori_loop(0, C // 16, cchunk, 0)
                return carry2

            lax.fori_loop(0, G // 16, rowchunk, 0)
            pltpu.sync_copy(rows.at[0], out.at[b, pl.ds(g * G, G)])
            return carry

        lax.fori_loop(wid * per_w, (wid + 1) * per_w, grp, 0)

    return k


def _sc_comp(V, RR, G):
    C = 16
    T = 3 * SEQ
    NG = V // G
    gpw = NG // NW
    mesh = plsc.VectorSubcoreMesh(core_axis_name="c", subcore_axis_name="s",
                                  num_cores=NC, num_subcores=NS)

    @functools.partial(
        pl.kernel, mesh=mesh,
        out_type=jax.ShapeDtypeStruct((B, V, 3), jnp.float32),
        compiler_params=pltpu.CompilerParams(use_tc_tiling_on_sc=False,
                                             needs_layout_passes=False),
        scratch_types=[
            pltpu.VMEM((SEQ, G), jnp.int32),
            pltpu.VMEM((SEQ, G, 16), jnp.float32),
            pltpu.VMEM((T, G), jnp.int32),
            pltpu.VMEM((2, T, G), jnp.int32),
            pltpu.VMEM((T, G), jnp.float32),
            pltpu.VMEM((2, T, G, C), jnp.float32),
            pltpu.VMEM((2, G, 3), jnp.float32),
            pltpu.VMEM((C,), jnp.float32),
            pltpu.SemaphoreType.DMA,
            pltpu.SemaphoreType.DMA,
        ],
    )
    def k(src, sg, uiw, bias, out, sidxv, uiwg, idxb, idxv, wv, rows, obuf,
          biasv, sem, osem):
        wid = lax.axis_index("s") * NC + lax.axis_index("c")
        pltpu.sync_copy(bias, biasv)
        lane = lax.iota(jnp.int32, 16)
        omask = lane < 3

        def grp(gi, carry):
            pltpu.sync_copy(sg.at[gi], sidxv)
            cps = [pltpu.async_copy(uiw.at[sidxv.at[s]], uiwg.at[s], sem)
                   for s in range(SEQ)]
            for cp in cps:
                cp.wait()

            def compose(rc, carry2):
                rids = lane + rc * 16
                for s in range(SEQ):
                    svec = jnp.full((16,), s, jnp.int32)
                    for kk in range(3):
                        t = s * 3 + kk
                        iv = plsc.load_gather(
                            uiwg, [svec, rids, jnp.full((16,), kk, jnp.int32)])
                        idxb[t, pl.ds(rc * 16, 16)] = iv.astype(jnp.int32) + s
                        wvv = plsc.load_gather(
                            uiwg, [svec, rids, jnp.full((16,), 4 + kk, jnp.int32)])
                        wv[t, pl.ds(rc * 16, 16)] = wvv
                return carry2

            lax.fori_loop(0, G // 16, compose, 0)

            def fire(b, par):
                def off(rc, carry3):
                    cs = pl.ds(rc * 16, 16)
                    for t in range(T):
                        idxv[par, t, cs] = idxb[t, cs] + b * (RR * SEQ)
                    return carry3

                lax.fori_loop(0, G // 16, off, 0)
                return [pltpu.async_copy(src.at[idxv.at[par, t]],
                                         rows.at[par, t], sem)
                        for t in range(T)]

            pend = fire(0, 0)
            opend = [None, None]
            for b in range(B):
                par = b % 2
                for cp in pend:
                    cp.wait()
                if b + 1 < B:
                    pend = fire(b + 1, 1 - par)
                if opend[par] is not None:
                    opend[par].wait()

                def rowchunk(rc, carry3):
                    def taps(t, accs):
                        wreg = wv[t, pl.ds(rc * 16, 16)]
                        return [accs[j] + wreg[j] * rows[par, t, rc * 16 + j, :]
                                for j in range(16)]

                    accs = lax.fori_loop(0, T, taps, [biasv[:]] * 16)
                    for j in range(16):
                        plsc.store_scatter(
                            obuf.at[par],
                            [jnp.full((16,), rc * 16 + j, jnp.int32), lane],
                            accs[j], mask=omask)
                    return carry3

                lax.fori_loop(0, G // 16, rowchunk, 0)
                opend[par] = pltpu.async_copy(
                    obuf.at[par], out.at[b, pl.ds(gi * G, G)], osem)
            for h in opend:
                if h is not None:
                    h.wait()
            return carry

        lax.fori_loop(wid * gpw, (wid + 1) * gpw, grp, 0)

    return k


def _sc_spiral(V, C, G, act):
    NG = V // G
    per_w = (B * NG) // NW
    mesh = plsc.VectorSubcoreMesh(core_axis_name="c", subcore_axis_name="s",
                                  num_cores=NC, num_subcores=NS)

    @functools.partial(
        pl.kernel, mesh=mesh,
        out_type=jax.ShapeDtypeStruct((B, V, C), jnp.float32),
        scratch_types=[
            pltpu.VMEM((SEQ, G), jnp.int32),
            pltpu.VMEM((SEQ, G, C), jnp.float32),
            pltpu.VMEM((C,), jnp.float32),
            pltpu.SemaphoreType.DMA,
        ],
    )
    def k(src, idx, bias, out, idxv, rows, biasv, sem):
        wid = lax.axis_index("s") * NC + lax.axis_index("c")
        pltpu.sync_copy(bias, biasv)

        def grp(i, carry):
            b = i // NG
            g = i % NG
            pltpu.sync_copy(idx.at[b, g], idxv)
            cps = [pltpu.async_copy(src.at[idxv.at[t]], rows.at[t], sem)
                   for t in range(SEQ)]
            for cp in cps:
                cp.wait()

            def row(r, carry2):
                def cchunk(cc, carry3):
                    cs = pl.ds(cc * 16, 16)
                    acc = rows[0, r, cs]
                    for t in range(1, SEQ):
                        acc = acc + rows[t, r, cs]
                    acc = acc + biasv[cs]
                    if act:
                        acc = jnp.where(acc > 0, acc, jnp.exp(acc) - 1.0)
                    rows[0, r, cs] = acc
                    return carry3

                lax.fori_loop(0, C // 16, cchunk, 0)
                return carry2

            lax.fori_loop(0, G, row, 0)
            pltpu.sync_copy(rows.at[0], out.at[b, pl.ds(g * G, G)])
            return carry

        lax.fori_loop(wid * per_w, (wid + 1) * per_w, grp, 0)

    return k


def _prep_pool(ui, uw, R, G):
    V = ui.shape[0]
    NG = V // G
    idx = ui.astype(jnp.int32)[None] + (jnp.arange(B, dtype=jnp.int32) * R)[:, None, None]
    idxg = idx.reshape(B, NG, G, 3).transpose(0, 1, 3, 2)
    wg = uw.reshape(NG, G, 3).transpose(0, 2, 1)
    return idxg, wg


def _prep_spiral(S, G):
    V = S.shape[0]
    NG = V // G
    base = (S + (jnp.arange(SEQ, dtype=jnp.int32) * (B * V))[None, :]).astype(jnp.int32)
    idx = base[None] + (jnp.arange(B, dtype=jnp.int32) * V)[:, None, None]
    return idx.reshape(B, NG, G, SEQ).transpose(0, 1, 3, 2)


def _prep_comp(S, ui, uw, G):
    V, NG = S.shape[0], S.shape[0] // G
    ui16 = (ui.astype(jnp.int32) * SEQ).astype(jnp.float32)
    z1 = jnp.zeros((ui.shape[0], 1), jnp.float32)
    uiw = jnp.concatenate(
        [ui16, z1, uw, jnp.zeros((ui.shape[0], 9), jnp.float32)], axis=1)
    sg = S.astype(jnp.int32).reshape(NG, G, SEQ).transpose(0, 2, 1)
    return sg, uiw


def _w_all(W, C_in, C_out, pad_to=None):
    P = pad_to or C_out
    wa = W.reshape(C_out, SEQ, C_in).transpose(2, 1, 0)
    if P != C_out:
        wa = jnp.pad(wa, ((0, 0), (0, 0), (0, P - C_out)))
    return wa.reshape(C_in, SEQ * P)


def kernel(z, Wp, bp, W0, b0, W1, b1, W2, b2, uw0, uw1, uw2,
           S0, S1, S2, ui0, ui1, ui2):
    x0 = _mm_nt(z, Wp, bp.reshape(1, -1), 4096).reshape(B * 256, 256)

    idxA, wA = _prep_pool(ui2, uw2, 256, 64)
    pA = _sc_pool(1024, 256, 64)(x0, idxA, wA).reshape(B * 1024, 256)
    hA = _mm_smajor(pA, _w_all(W0, 256, 256), 2048).reshape(SEQ * B * 1024, 256)
    yA = _sc_spiral(1024, 256, 16, True)(hA, _prep_spiral(S2, 16), b0)

    idxB, wB = _prep_pool(ui1, uw1, 1024, 64)
    pB = _sc_pool(4096, 256, 64)(yA.reshape(B * 1024, 256), idxB, wB).reshape(B * 4096, 256)
    hB = _mm_smajor(pB, _w_all(W1, 256, 128), 4096).reshape(SEQ * B * 4096, 128)
    yB = _sc_spiral(4096, 128, 32, True)(hB, _prep_spiral(S1, 32), b1)

    hC = _mm(yB.reshape(B * 4096, 128), _w_all(W2, 128, 3, pad_to=16),
             2048, 256).reshape(B * 4096 * SEQ, 16)
    sgC, uiwC = _prep_comp(S0, ui0, uw0, 64)
    b2p = jnp.pad(b2, (0, 13))
    return _sc_comp(16384, 4096, 64)(hC, sgC, uiwC, b2p)


# speedup vs baseline: 5.8927x; 1.0909x over previous
import functools

import jax
import jax.numpy as jnp
from jax import lax
from jax.experimental import pallas as pl
from jax.experimental.pallas import tpu as pltpu
from jax.experimental.pallas import tpu_sc as plsc

SEQ = 16
B = 8
NC, NS = 2, 16
NW = NC * NS


def _mm(x, w, bm, bn):
    M, K = x.shape
    _, N = w.shape

    def body(xr, wr, outr):
        outr[...] = jnp.dot(xr[...], wr[...], preferred_element_type=jnp.float32)

    return pl.pallas_call(
        body,
        grid=(M // bm, N // bn),
        in_specs=[pl.BlockSpec((bm, K), lambda i, j: (i, 0)),
                  pl.BlockSpec((K, bn), lambda i, j: (0, j))],
        out_specs=pl.BlockSpec((bm, bn), lambda i, j: (i, j)),
        out_shape=jax.ShapeDtypeStruct((M, N), jnp.float32),
    )(x, w)


def _mm_smajor(x, w, bm):
    M, K = x.shape
    C = w.shape[1] // SEQ

    def body(xr, wr, outr):
        outr[0] = jnp.dot(xr[...], wr[...], preferred_element_type=jnp.float32)

    return pl.pallas_call(
        body,
        grid=(M // bm, SEQ),
        in_specs=[pl.BlockSpec((bm, K), lambda i, s: (i, 0)),
                  pl.BlockSpec((K, C), lambda i, s: (0, s))],
        out_specs=pl.BlockSpec((1, bm, C), lambda i, s: (s, i, 0)),
        out_shape=jax.ShapeDtypeStruct((SEQ, M, C), jnp.float32),
    )(x, w)


def _mm_nt(x, w, bias, bn):
    M, K = x.shape
    N = w.shape[0]

    def body(xr, wr, br, outr):
        acc = lax.dot_general(xr[...], wr[...], (((1,), (1,)), ((), ())),
                              preferred_element_type=jnp.float32)
        outr[...] = acc + br[...]

    return pl.pallas_call(
        body,
        grid=(N // bn,),
        in_specs=[pl.BlockSpec((M, K), lambda j: (0, 0)),
                  pl.BlockSpec((bn, K), lambda j: (j, 0)),
                  pl.BlockSpec((1, bn), lambda j: (0, j))],
        out_specs=pl.BlockSpec((M, bn), lambda j: (0, j)),
        out_shape=jax.ShapeDtypeStruct((M, N), jnp.float32),
    )(x, w, bias)


def _sc_pool(V, C, G, nb):
    NG = V // G
    per_w = (nb * NG) // NW
    mesh = plsc.VectorSubcoreMesh(core_axis_name="c", subcore_axis_name="s",
                                  num_cores=NC, num_subcores=NS)

    @functools.partial(
        pl.kernel, mesh=mesh,
        out_type=jax.ShapeDtypeStruct((nb, V, C), jnp.float32),
        scratch_types=[
            pltpu.VMEM((3, G), jnp.int32),
            pltpu.VMEM((3, G), jnp.float32),
            pltpu.VMEM((3, G, C), jnp.float32),
            pltpu.SemaphoreType.DMA,
        ],
    )
    def k(src, idx, w, out, idxv, wv, rows, sem):
        wid = lax.axis_index("s") * NC + lax.axis_index("c")

        def grp(i, carry):
            b = i // NG
            g = i % NG
            pltpu.sync_copy(idx.at[b, g], idxv)
            pltpu.sync_copy(w.at[g], wv)
            cps = [pltpu.async_copy(src.at[idxv.at[t]], rows.at[t], sem)
                   for t in range(3)]
            for cp in cps:
                cp.wait()

            def rowchunk(rc, carry2):
                wr = [wv[t, pl.ds(rc * 16, 16)] for t in range(3)]
                for j in range(16):
                    r = rc * 16 + j
                    w0, w1, w2 = wr[0][j], wr[1][j], wr[2][j]

                    def cchunk(cc, carry3):
                        cs = pl.ds(cc * 16, 16)
                        acc = (w0 * rows[0, r, cs] + w1 * rows[1, r, cs]
                               + w2 * rows[2, r, cs])
                        rows[0, r, cs] = acc
                        return carry3

                    lax.fori_loop(0, C // 16, cchunk, 0)
                return carry2

            lax.fori_loop(0, G // 16, rowchunk, 0)
            pltpu.sync_copy(rows.at[0], out.at[b, pl.ds(g * G, G)])
            return carry

        lax.fori_loop(wid * per_w, (wid + 1) * per_w, grp, 0)

    return k


def _sc_comp(V, RR, G, nb):
    C = 16
    T = 3 * SEQ
    NG = V // G
    gpw = NG // NW
    mesh = plsc.VectorSubcoreMesh(core_axis_name="c", subcore_axis_name="s",
                                  num_cores=NC, num_subcores=NS)

    @functools.partial(
        pl.kernel, mesh=mesh,
        out_type=jax.ShapeDtypeStruct((nb, V, 3), jnp.float32),
        compiler_params=pltpu.CompilerParams(use_tc_tiling_on_sc=False,
                                             needs_layout_passes=False),
        scratch_types=[
            pltpu.VMEM((SEQ, G), jnp.int32),
            pltpu.VMEM((SEQ, G, 16), jnp.float32),
            pltpu.VMEM((T, G), jnp.int32),
            pltpu.VMEM((2, T, G), jnp.int32),
            pltpu.VMEM((T, G), jnp.float32),
            pltpu.VMEM((2, T, G, C), jnp.float32),
            pltpu.VMEM((2, G, 3), jnp.float32),
            pltpu.VMEM((C,), jnp.float32),
            pltpu.SemaphoreType.DMA,
            pltpu.SemaphoreType.DMA,
        ],
    )
    def k(src, sg, uiw, bias, out, sidxv, uiwg, idxb, idxv, wv, rows, obuf,
          biasv, sem, osem):
        wid = lax.axis_index("s") * NC + lax.axis_index("c")
        pltpu.sync_copy(bias, biasv)
        lane = lax.iota(jnp.int32, 16)
        omask = lane < 3

        def grp(gi, carry):
            pltpu.sync_copy(sg.at[gi], sidxv)
            cps = [pltpu.async_copy(uiw.at[sidxv.at[s]], uiwg.at[s], sem)
                   for s in range(SEQ)]
            for cp in cps:
                cp.wait()

            def compose(rc, carry2):
                rids = lane + rc * 16
                for s in range(SEQ):
                    svec = jnp.full((16,), s, jnp.int32)
                    for kk in range(3):
                        t = s * 3 + kk
                        iv = plsc.load_gather(
                            uiwg, [svec, rids, jnp.full((16,), kk, jnp.int32)])
                        idxb[t, pl.ds(rc * 16, 16)] = iv.astype(jnp.int32) + s
                        wvv = plsc.load_gather(
                            uiwg, [svec, rids, jnp.full((16,), 4 + kk, jnp.int32)])
                        wv[t, pl.ds(rc * 16, 16)] = wvv
                return carry2

            lax.fori_loop(0, G // 16, compose, 0)

            def fire(b, par):
                def off(rc, carry3):
                    cs = pl.ds(rc * 16, 16)
                    for t in range(T):
                        idxv[par, t, cs] = idxb[t, cs] + b * (RR * SEQ)
                    return carry3

                lax.fori_loop(0, G // 16, off, 0)
                return [pltpu.async_copy(src.at[idxv.at[par, t]],
                                         rows.at[par, t], sem)
                        for t in range(T)]

            pend = fire(0, 0)
            opend = [None, None]
            for b in range(nb):
                par = b % 2
                for cp in pend:
                    cp.wait()
                if b + 1 < nb:
                    pend = fire(b + 1, 1 - par)
                if opend[par] is not None:
                    opend[par].wait()

                def rowchunk(rc, carry3):
                    def taps(t, accs):
                        wreg = wv[t, pl.ds(rc * 16, 16)]
                        return [accs[j] + wreg[j] * rows[par, t, rc * 16 + j, :]
                                for j in range(16)]

                    accs = lax.fori_loop(0, T, taps, [biasv[:]] * 16)
                    for j in range(16):
                        plsc.store_scatter(
                            obuf.at[par],
                            [jnp.full((16,), rc * 16 + j, jnp.int32), lane],
                            accs[j], mask=omask)
                    return carry3

                lax.fori_loop(0, G // 16, rowchunk, 0)
                opend[par] = pltpu.async_copy(
                    obuf.at[par], out.at[b, pl.ds(gi * G, G)], osem)
            for h in opend:
                if h is not None:
                    h.wait()
            return carry

        lax.fori_loop(wid * gpw, (wid + 1) * gpw, grp, 0)

    return k


def _sc_spiral(V, C, G, act, nb):
    NG = V // G
    per_w = (nb * NG) // NW
    mesh = plsc.VectorSubcoreMesh(core_axis_name="c", subcore_axis_name="s",
                                  num_cores=NC, num_subcores=NS)

    @functools.partial(
        pl.kernel, mesh=mesh,
        out_type=jax.ShapeDtypeStruct((nb, V, C), jnp.float32),
        scratch_types=[
            pltpu.VMEM((SEQ, G), jnp.int32),
            pltpu.VMEM((SEQ, G, C), jnp.float32),
            pltpu.VMEM((C,), jnp.float32),
            pltpu.SemaphoreType.DMA,
        ],
    )
    def k(src, idx, bias, out, idxv, rows, biasv, sem):
        wid = lax.axis_index("s") * NC + lax.axis_index("c")
        pltpu.sync_copy(bias, biasv)

        def grp(i, carry):
            b = i // NG
            g = i % NG
            pltpu.sync_copy(idx.at[b, g], idxv)
            cps = [pltpu.async_copy(src.at[idxv.at[t]], rows.at[t], sem)
                   for t in range(SEQ)]
            for cp in cps:
                cp.wait()

            def row(r, carry2):
                def cchunk(cc, carry3):
                    cs = pl.ds(cc * 16, 16)
                    acc = rows[0, r, cs]
                    for t in range(1, SEQ):
                        acc = acc + rows[t, r, cs]
                    acc = acc + biasv[cs]
                    if act:
                        acc = jnp.where(acc > 0, acc, jnp.exp(acc) - 1.0)
                    rows[0, r, cs] = acc
                    return carry3

                lax.fori_loop(0, C // 16, cchunk, 0)
                return carry2

            lax.fori_loop(0, G, row, 0)
            pltpu.sync_copy(rows.at[0], out.at[b, pl.ds(g * G, G)])
            return carry

        lax.fori_loop(wid * per_w, (wid + 1) * per_w, grp, 0)

    return k


def _prep_pool(ui, uw, R, G, nb):
    V = ui.shape[0]
    NG = V // G
    idx = ui.astype(jnp.int32)[None] + (jnp.arange(nb, dtype=jnp.int32) * R)[:, None, None]
    idxg = idx.reshape(nb, NG, G, 3).transpose(0, 1, 3, 2)
    wg = uw.reshape(NG, G, 3).transpose(0, 2, 1)
    return idxg, wg


def _prep_spiral(S, G, nb):
    V = S.shape[0]
    NG = V // G
    base = (S + (jnp.arange(SEQ, dtype=jnp.int32) * (nb * V))[None, :]).astype(jnp.int32)
    idx = base[None] + (jnp.arange(nb, dtype=jnp.int32) * V)[:, None, None]
    return idx.reshape(nb, NG, G, SEQ).transpose(0, 1, 3, 2)


def _prep_comp(S, ui, uw, G):
    V, NG = S.shape[0], S.shape[0] // G
    ui16 = (ui.astype(jnp.int32) * SEQ).astype(jnp.float32)
    z1 = jnp.zeros((ui.shape[0], 1), jnp.float32)
    uiw = jnp.concatenate(
        [ui16, z1, uw, jnp.zeros((ui.shape[0], 9), jnp.float32)], axis=1)
    sg = S.astype(jnp.int32).reshape(NG, G, SEQ).transpose(0, 2, 1)
    return sg, uiw


def _w_all(W, C_in, C_out, pad_to=None):
    P = pad_to or C_out
    wa = W.reshape(C_out, SEQ, C_in).transpose(2, 1, 0)
    if P != C_out:
        wa = jnp.pad(wa, ((0, 0), (0, 0), (0, P - C_out)))
    return wa.reshape(C_in, SEQ * P)


def kernel(z, Wp, bp, W0, b0, W1, b1, W2, b2, uw0, uw1, uw2,
           S0, S1, S2, ui0, ui1, ui2):
    NB = B // 2

    x0 = _mm_nt(z, Wp, bp.reshape(1, -1), 4096).reshape(B * 256, 256)

    idxA, wA = _prep_pool(ui2, uw2, 256, 64, NB)
    sidxA = _prep_spiral(S2, 16, NB)
    wallA = _w_all(W0, 256, 256)
    idxB, wB = _prep_pool(ui1, uw1, 1024, 64, NB)
    sidxB = _prep_spiral(S1, 32, NB)
    wallB = _w_all(W1, 256, 128)
    wallC = _w_all(W2, 128, 3, pad_to=16)
    sgC, uiwC = _prep_comp(S0, ui0, uw0, 64)
    b2p = jnp.pad(b2, (0, 13))

    pool_a = _sc_pool(1024, 256, 64, NB)
    spiral_a = _sc_spiral(1024, 256, 16, True, NB)
    pool_b = _sc_pool(4096, 256, 64, NB)
    spiral_b = _sc_spiral(4096, 128, 32, True, NB)
    comp_c = _sc_comp(16384, 4096, 64, NB)

    outs = []
    for h in range(2):
        xh = x0[h * NB * 256:(h + 1) * NB * 256]
        pA = pool_a(xh, idxA, wA).reshape(NB * 1024, 256)
        hA = _mm_smajor(pA, wallA, 2048).reshape(SEQ * NB * 1024, 256)
        yA = spiral_a(hA, sidxA, b0)
        pB = pool_b(yA.reshape(NB * 1024, 256), idxB, wB).reshape(NB * 4096, 256)
        hB = _mm_smajor(pB, wallB, 4096).reshape(SEQ * NB * 4096, 128)
        yB = spiral_b(hB, sidxB, b1)
        hC = _mm(yB.reshape(NB * 4096, 128), wallC,
                 2048, 256).reshape(NB * 4096 * SEQ, 16)
        outs.append(comp_c(hC, sgC, uiwC, b2p))
    return jnp.concatenate(outs, axis=0)


# speedup vs baseline: 6.2027x; 1.0526x over previous
import functools

import jax
import jax.numpy as jnp
from jax import lax
from jax.experimental import pallas as pl
from jax.experimental.pallas import tpu as pltpu
from jax.experimental.pallas import tpu_sc as plsc

SEQ = 16
B = 8
NC, NS = 2, 16
NW = NC * NS


def _mm(x, w, bm, bn):
    M, K = x.shape
    _, N = w.shape

    def body(xr, wr, outr):
        outr[...] = jnp.dot(xr[...], wr[...], preferred_element_type=jnp.float32)

    return pl.pallas_call(
        body,
        grid=(M // bm, N // bn),
        in_specs=[pl.BlockSpec((bm, K), lambda i, j: (i, 0)),
                  pl.BlockSpec((K, bn), lambda i, j: (0, j))],
        out_specs=pl.BlockSpec((bm, bn), lambda i, j: (i, j)),
        out_shape=jax.ShapeDtypeStruct((M, N), jnp.float32),
    )(x, w)


def _mm_smajor(x, w, bm):
    M, K = x.shape
    C = w.shape[1] // SEQ

    def body(xr, wr, outr):
        outr[0] = jnp.dot(xr[...], wr[...], preferred_element_type=jnp.float32)

    return pl.pallas_call(
        body,
        grid=(M // bm, SEQ),
        in_specs=[pl.BlockSpec((bm, K), lambda i, s: (i, 0)),
                  pl.BlockSpec((K, C), lambda i, s: (0, s))],
        out_specs=pl.BlockSpec((1, bm, C), lambda i, s: (s, i, 0)),
        out_shape=jax.ShapeDtypeStruct((SEQ, M, C), jnp.float32),
    )(x, w)


def _mm_nt(x, w, bias, bn):
    M, K = x.shape
    N = w.shape[0]

    def body(xr, wr, br, outr):
        acc = lax.dot_general(xr[...], wr[...], (((1,), (1,)), ((), ())),
                              preferred_element_type=jnp.float32)
        outr[...] = acc + br[...]

    return pl.pallas_call(
        body,
        grid=(N // bn,),
        in_specs=[pl.BlockSpec((M, K), lambda j: (0, 0)),
                  pl.BlockSpec((bn, K), lambda j: (j, 0)),
                  pl.BlockSpec((1, bn), lambda j: (0, j))],
        out_specs=pl.BlockSpec((M, bn), lambda j: (0, j)),
        out_shape=jax.ShapeDtypeStruct((M, N), jnp.float32),
    )(x, w, bias)


def _sc_pool(V, C, G, nb):
    NG = V // G
    per_w = (nb * NG) // NW
    mesh = plsc.VectorSubcoreMesh(core_axis_name="c", subcore_axis_name="s",
                                  num_cores=NC, num_subcores=NS)

    @functools.partial(
        pl.kernel, mesh=mesh,
        out_type=jax.ShapeDtypeStruct((nb, V, C), jnp.float32),
        scratch_types=[
            pltpu.VMEM((3, G), jnp.int32),
            pltpu.VMEM((3, G), jnp.float32),
            pltpu.VMEM((3, G, C), jnp.float32),
            pltpu.SemaphoreType.DMA,
        ],
    )
    def k(src, idx, w, out, idxv, wv, rows, sem):
        wid = lax.axis_index("s") * NC + lax.axis_index("c")

        def grp(i, carry):
            b = i // NG
            g = i % NG
            pltpu.sync_copy(idx.at[b, g], idxv)
            pltpu.sync_copy(w.at[g], wv)
            cps = [pltpu.async_copy(src.at[idxv.at[t]], rows.at[t], sem)
                   for t in range(3)]
            for cp in cps:
                cp.wait()

            def rowchunk(rc, carry2):
                wr = [wv[t, pl.ds(rc * 16, 16)] for t in range(3)]
                for j in range(16):
                    r = rc * 16 + j
                    w0, w1, w2 = wr[0][j], wr[1][j], wr[2][j]

                    def cchunk(cc, carry3):
                        cs = pl.ds(cc * 16, 16)
                        acc = (w0 * rows[0, r, cs] + w1 * rows[1, r, cs]
                               + w2 * rows[2, r, cs])
                        rows[0, r, cs] = acc
                        return carry3

                    lax.fori_loop(0, C // 16, cchunk, 0)
                return carry2

            lax.fori_loop(0, G // 16, rowchunk, 0)
            pltpu.sync_copy(rows.at[0], out.at[b, pl.ds(g * G, G)])
            return carry

        lax.fori_loop(wid * per_w, (wid + 1) * per_w, grp, 0)

    return k


def _sc_comp(V, RR, G, nb):
    C = 16
    T = 3 * SEQ
    NG = V // G
    gpw = NG // NW
    mesh = plsc.VectorSubcoreMesh(core_axis_name="c", subcore_axis_name="s",
                                  num_cores=NC, num_subcores=NS)

    @functools.partial(
        pl.kernel, mesh=mesh,
        out_type=jax.ShapeDtypeStruct((nb, V, 3), jnp.float32),
        compiler_params=pltpu.CompilerParams(use_tc_tiling_on_sc=False,
                                             needs_layout_passes=False),
        scratch_types=[
            pltpu.VMEM((SEQ, G), jnp.int32),
            pltpu.VMEM((SEQ, G, 16), jnp.float32),
            pltpu.VMEM((T, G), jnp.int32),
            pltpu.VMEM((2, T, G), jnp.int32),
            pltpu.VMEM((T, G), jnp.float32),
            pltpu.VMEM((2, T, G, C), jnp.float32),
            pltpu.VMEM((2, G, 3), jnp.float32),
            pltpu.VMEM((C,), jnp.float32),
            pltpu.SemaphoreType.DMA,
            pltpu.SemaphoreType.DMA,
        ],
    )
    def k(src, sg, uiw, bias, out, sidxv, uiwg, idxb, idxv, wv, rows, obuf,
          biasv, sem, osem):
        wid = lax.axis_index("s") * NC + lax.axis_index("c")
        pltpu.sync_copy(bias, biasv)
        lane = lax.iota(jnp.int32, 16)
        omask = lane < 3

        def grp(gi, carry):
            pltpu.sync_copy(sg.at[gi], sidxv)
            cps = [pltpu.async_copy(uiw.at[sidxv.at[s]], uiwg.at[s], sem)
                   for s in range(SEQ)]
            for cp in cps:
                cp.wait()

            def compose(rc, carry2):
                rids = lane + rc * 16
                for s in range(SEQ):
                    svec = jnp.full((16,), s, jnp.int32)
                    for kk in range(3):
                        t = s * 3 + kk
                        iv = plsc.load_gather(
                            uiwg, [svec, rids, jnp.full((16,), kk, jnp.int32)])
                        idxb[t, pl.ds(rc * 16, 16)] = iv.astype(jnp.int32) + s
                        wvv = plsc.load_gather(
                            uiwg, [svec, rids, jnp.full((16,), 4 + kk, jnp.int32)])
                        wv[t, pl.ds(rc * 16, 16)] = wvv
                return carry2

            lax.fori_loop(0, G // 16, compose, 0)

            def fire(b, par):
                def off(rc, carry3):
                    cs = pl.ds(rc * 16, 16)
                    for t in range(T):
                        idxv[par, t, cs] = idxb[t, cs] + b * (RR * SEQ)
                    return carry3

                lax.fori_loop(0, G // 16, off, 0)
                return [pltpu.async_copy(src.at[idxv.at[par, t]],
                                         rows.at[par, t], sem)
                        for t in range(T)]

            pend = fire(0, 0)
            opend = [None, None]
            for b in range(nb):
                par = b % 2
                for cp in pend:
                    cp.wait()
                if b + 1 < nb:
                    pend = fire(b + 1, 1 - par)
                if opend[par] is not None:
                    opend[par].wait()

                def rowchunk(rc, carry3):
                    def taps(t, accs):
                        wreg = wv[t, pl.ds(rc * 16, 16)]
                        return [accs[j] + wreg[j] * rows[par, t, rc * 16 + j, :]
                                for j in range(16)]

                    accs = lax.fori_loop(0, T, taps, [biasv[:]] * 16)
                    for j in range(16):
                        plsc.store_scatter(
                            obuf.at[par],
                            [jnp.full((16,), rc * 16 + j, jnp.int32), lane],
                            accs[j], mask=omask)
                    return carry3

                lax.fori_loop(0, G // 16, rowchunk, 0)
                opend[par] = pltpu.async_copy(
                    obuf.at[par], out.at[b, pl.ds(gi * G, G)], osem)
            for h in opend:
                if h is not None:
                    h.wait()
            return carry

        lax.fori_loop(wid * gpw, (wid + 1) * gpw, grp, 0)

    return k


def _sc_spiral(V, C, G, act, nb):
    NG = V // G
    per_w = (nb * NG) // NW
    mesh = plsc.VectorSubcoreMesh(core_axis_name="c", subcore_axis_name="s",
                                  num_cores=NC, num_subcores=NS)

    @functools.partial(
        pl.kernel, mesh=mesh,
        out_type=jax.ShapeDtypeStruct((nb, V, C), jnp.float32),
        scratch_types=[
            pltpu.VMEM((2, SEQ, G), jnp.int32),
            pltpu.VMEM((2, SEQ, G, C), jnp.float32),
            pltpu.VMEM((2, G, C), jnp.float32),
            pltpu.VMEM((C,), jnp.float32),
            pltpu.SemaphoreType.DMA,
            pltpu.SemaphoreType.DMA,
        ],
    )
    def k(src, idx, bias, out, idxv, rows, obuf, biasv, sem, osem):
        wid = lax.axis_index("s") * NC + lax.axis_index("c")
        pltpu.sync_copy(bias, biasv)
        base = wid * per_w

        def fire(i, par):
            b = i // NG
            g = i % NG
            pltpu.sync_copy(idx.at[b, g], idxv.at[par])
            return [pltpu.async_copy(src.at[idxv.at[par, t]], rows.at[par, t],
                                     sem)
                    for t in range(SEQ)]

        pend = fire(base, 0)
        opend = [None, None]
        for st in range(per_w):
            par = st % 2
            i = base + st
            for cp in pend:
                cp.wait()
            if st + 1 < per_w:
                pend = fire(i + 1, 1 - par)
            if opend[par] is not None:
                opend[par].wait()

            def row(r, carry2):
                def cchunk(cc, carry3):
                    cs = pl.ds(cc * 16, 16)
                    acc = rows[par, 0, r, cs]
                    for t in range(1, SEQ):
                        acc = acc + rows[par, t, r, cs]
                    acc = acc + biasv[cs]
                    if act:
                        acc = jnp.where(acc > 0, acc, jnp.exp(acc) - 1.0)
                    obuf[par, r, cs] = acc
                    return carry3

                lax.fori_loop(0, C // 16, cchunk, 0)
                return carry2

            lax.fori_loop(0, G, row, 0)
            opend[par] = pltpu.async_copy(
                obuf.at[par], out.at[i // NG, pl.ds((i % NG) * G, G)], osem)
        for h in opend:
            if h is not None:
                h.wait()

    return k


def _prep_pool(ui, uw, R, G, nb):
    V = ui.shape[0]
    NG = V // G
    idx = ui.astype(jnp.int32)[None] + (jnp.arange(nb, dtype=jnp.int32) * R)[:, None, None]
    idxg = idx.reshape(nb, NG, G, 3).transpose(0, 1, 3, 2)
    wg = uw.reshape(NG, G, 3).transpose(0, 2, 1)
    return idxg, wg


def _prep_spiral(S, G, nb):
    V = S.shape[0]
    NG = V // G
    base = (S + (jnp.arange(SEQ, dtype=jnp.int32) * (nb * V))[None, :]).astype(jnp.int32)
    idx = base[None] + (jnp.arange(nb, dtype=jnp.int32) * V)[:, None, None]
    return idx.reshape(nb, NG, G, SEQ).transpose(0, 1, 3, 2)


def _prep_comp(S, ui, uw, G):
    V, NG = S.shape[0], S.shape[0] // G
    ui16 = (ui.astype(jnp.int32) * SEQ).astype(jnp.float32)
    z1 = jnp.zeros((ui.shape[0], 1), jnp.float32)
    uiw = jnp.concatenate(
        [ui16, z1, uw, jnp.zeros((ui.shape[0], 9), jnp.float32)], axis=1)
    sg = S.astype(jnp.int32).reshape(NG, G, SEQ).transpose(0, 2, 1)
    return sg, uiw


def _w_all(W, C_in, C_out, pad_to=None):
    P = pad_to or C_out
    wa = W.reshape(C_out, SEQ, C_in).transpose(2, 1, 0)
    if P != C_out:
        wa = jnp.pad(wa, ((0, 0), (0, 0), (0, P - C_out)))
    return wa.reshape(C_in, SEQ * P)


def kernel(z, Wp, bp, W0, b0, W1, b1, W2, b2, uw0, uw1, uw2,
           S0, S1, S2, ui0, ui1, ui2):
    NB = B // 2

    x0 = _mm_nt(z, Wp, bp.reshape(1, -1), 4096).reshape(B * 256, 256)

    idxA, wA = _prep_pool(ui2, uw2, 256, 64, NB)
    sidxA = _prep_spiral(S2, 8, NB)
    wallA = _w_all(W0, 256, 256)
    idxB, wB = _prep_pool(ui1, uw1, 1024, 64, NB)
    sidxB = _prep_spiral(S1, 16, NB)
    wallB = _w_all(W1, 256, 128)
    wallC = _w_all(W2, 128, 3, pad_to=16)
    sgC, uiwC = _prep_comp(S0, ui0, uw0, 64)
    b2p = jnp.pad(b2, (0, 13))

    pool_a = _sc_pool(1024, 256, 64, NB)
    spiral_a = _sc_spiral(1024, 256, 8, True, NB)
    pool_b = _sc_pool(4096, 256, 64, NB)
    spiral_b = _sc_spiral(4096, 128, 16, True, NB)
    comp_c = _sc_comp(16384, 4096, 64, NB)

    outs = []
    for h in range(2):
        xh = x0[h * NB * 256:(h + 1) * NB * 256]
        pA = pool_a(xh, idxA, wA).reshape(NB * 1024, 256)
        hA = _mm_smajor(pA, wallA, 2048).reshape(SEQ * NB * 1024, 256)
        yA = spiral_a(hA, sidxA, b0)
        pB = pool_b(yA.reshape(NB * 1024, 256), idxB, wB).reshape(NB * 4096, 256)
        hB = _mm_smajor(pB, wallB, 4096).reshape(SEQ * NB * 4096, 128)
        yB = spiral_b(hB, sidxB, b1)
        hC = _mm(yB.reshape(NB * 4096, 128), wallC,
                 2048, 256).reshape(NB * 4096 * SEQ, 16)
        outs.append(comp_c(hC, sgC, uiwC, b2p))
    return jnp.concatenate(outs, axis=0)
